# Initial kernel scaffold; baseline (speedup 1.0000x reference)
import jax
import jax.numpy as jnp
from jax.experimental import pallas as pl


def kernel(x, table, W, b):
    raise NotImplementedError("write your pallas kernel here")


# speedup vs baseline: 7.4371x; 7.4371x over previous
import functools

import jax
import jax.numpy as jnp
from jax import lax
from jax.experimental import pallas as pl
from jax.experimental.pallas import tpu as pltpu
from jax.experimental.pallas import tpu_sc as plsc

V = 1000000
D = 16
B = 16384
L = 200
E = B * L


_R = V // 8
_BLK_R = 1000


def _scores_body(t_ref, w_ref, s_ref):
    kk = lax.broadcasted_iota(jnp.int32, (8 * D, 8), 0)
    jj = lax.broadcasted_iota(jnp.int32, (8 * D, 8), 1)
    seg = (kk // D == jj).astype(jnp.float32)
    s_ref[...] = jnp.dot(
        t_ref[...] * w_ref[...], seg, preferred_element_type=jnp.float32
    )


def _compute_scores(table2, wt):
    return pl.pallas_call(
        _scores_body,
        grid=(_R // _BLK_R,),
        in_specs=[
            pl.BlockSpec((_BLK_R, 8 * D), lambda i: (i, 0)),
            pl.BlockSpec((1, 8 * D), lambda i: (0, 0)),
        ],
        out_specs=pl.BlockSpec((_BLK_R, 8), lambda i: (i, 0)),
        out_shape=jax.ShapeDtypeStruct((_R, 8), jnp.float32),
    )(table2, wt)


_NC = 2
_NS = 16
_NW = _NC * _NS
_PER_W = E // _NW
_CH = 25600
_NCHUNK = _PER_W // _CH


def _make_gather():
    mesh = plsc.VectorSubcoreMesh(core_axis_name="c", subcore_axis_name="s")

    @functools.partial(
        pl.kernel,
        out_type=jax.ShapeDtypeStruct((E,), jnp.float32),
        mesh=mesh,
        scratch_types=[
            pltpu.VMEM((_CH,), jnp.int32),
            pltpu.VMEM((_CH,), jnp.float32),
            pltpu.SemaphoreType.DMA,
        ],
    )
    def gather_kernel(xf_hbm, scores_hbm, out_hbm, idx_v, vals_v, sem):
        wid = lax.axis_index("s") * _NC + lax.axis_index("c")
        base = wid * _PER_W
        for c in range(_NCHUNK):
            off = base + c * _CH
            pltpu.sync_copy(xf_hbm.at[pl.ds(off, _CH)], idx_v)
            pltpu.async_copy(scores_hbm.at[idx_v], vals_v, sem).wait()
            pltpu.sync_copy(vals_v, out_hbm.at[pl.ds(off, _CH)])

    return gather_kernel


_gather = _make_gather()


_BLK_B = 2048


def _pool_body(v_ref, b_ref, o_ref):
    s = jnp.sum(v_ref[...], axis=1, keepdims=True) * (1.0 / L)
    z = s + b_ref[0]
    em = jnp.exp(-jnp.abs(z))
    p = jnp.where(z >= 0, 1.0, em) / (1.0 + em)
    o_ref[...] = jnp.round(p * 1e4) * 1e-4


def _pool(vals2d, b):
    return pl.pallas_call(
        _pool_body,
        grid=(B // _BLK_B,),
        in_specs=[
            pl.BlockSpec((_BLK_B, L), lambda i: (i, 0)),
            pl.BlockSpec(memory_space=pltpu.SMEM),
        ],
        out_specs=pl.BlockSpec((_BLK_B, 1), lambda i: (i, 0)),
        out_shape=jax.ShapeDtypeStruct((B, 1), jnp.float32),
    )(vals2d, b)


def kernel(x, table, W, b):
    xf = x.reshape(E).astype(jnp.int32)
    wt = jnp.tile(W.reshape(D), 8).reshape(1, 8 * D)
    scores = _compute_scores(table.reshape(_R, 8 * D), wt).reshape(V)
    vals = _gather(xf, scores)
    return _pool(vals.reshape(B, L), b)


# speedup vs baseline: 60.7361x; 8.1667x over previous
import functools

import jax
import jax.numpy as jnp
from jax import lax
from jax.experimental import pallas as pl
from jax.experimental.pallas import tpu as pltpu
from jax.experimental.pallas import tpu_sc as plsc

V = 1000000
D = 16
B = 16384
L = 200
E = B * L


_BLK_V = 32768


def _scores_body(t_ref, w_ref, b_ref, s_ref):
    s = lax.dot_general(
        w_ref[...],
        t_ref[...],
        (((1,), (0,)), ((), ())),
        preferred_element_type=jnp.float32,
    )
    s_ref[...] = s[0] + b_ref[0]


def _compute_scores(tableT, W, b):
    return pl.pallas_call(
        _scores_body,
        grid=(pl.cdiv(V, _BLK_V),),
        in_specs=[
            pl.BlockSpec((D, _BLK_V), lambda i: (0, i)),
            pl.BlockSpec((1, D), lambda i: (0, 0)),
            pl.BlockSpec(memory_space=pltpu.SMEM),
        ],
        out_specs=pl.BlockSpec((_BLK_V,), lambda i: (i,)),
        out_shape=jax.ShapeDtypeStruct((V,), jnp.float32),
    )(tableT, W, b)


_NC = 2
_NS = 16
_NW = _NC * _NS
_RPW = B // _NW
_CCH = 128
_CH = L * _CCH
_NCHUNK = _RPW // _CCH


def _make_gather_pool():
    mesh = plsc.VectorSubcoreMesh(core_axis_name="c", subcore_axis_name="s")

    LH0 = 104
    LH1 = L - LH0
    _LH = (LH0, LH1)
    st_per_tile = 62496
    st_sizes = [13312] * 4 + [9248]
    st_offs = [13312 * i for i in range(4)] + [53248]
    st_tail = V - 16 * st_per_tile

    @functools.partial(
        pl.kernel,
        out_type=jax.ShapeDtypeStruct((B,), jnp.float32),
        mesh=mesh,
        scratch_types=[
            pltpu.VMEM((LH0, _CCH), jnp.int32),
            pltpu.VMEM((LH0 * _CCH,), jnp.int32),
            pltpu.VMEM((LH1 * _CCH,), jnp.int32),
            pltpu.VMEM((LH0 * _CCH,), jnp.float32),
            pltpu.VMEM((LH0 * _CCH,), jnp.float32),
            pltpu.VMEM((_RPW,), jnp.float32),
            pltpu.MemorySpace.VMEM_SHARED((V,), jnp.float32),
            pltpu.SemaphoreType.DMA,
            pltpu.SemaphoreType.DMA,
            pltpu.SemaphoreType.DMA,
        ],
    )
    def gather_pool(
        xT_hbm, scores_hbm, out_hbm,
        idx2, idx1h0, idx1h1, val0, val1, outv, sscores,
        si, sg0, sg1,
    ):
        sid = lax.axis_index("s")
        wid = sid * _NC + lax.axis_index("c")
        col0 = wid * _RPW
        idx1h = (idx1h0, idx1h1)
        vals = (val0, val1)
        sg = (sg0, sg1)

        def idx_dma(ch, h):
            c = col0 + ch * _CCH
            l0 = 0 if h == 0 else LH0
            return pltpu.make_async_copy(
                xT_hbm.at[pl.ds(l0, _LH[h]), pl.ds(c, _CCH)],
                idx2.at[pl.ds(0, _LH[h])] if h == 1 else idx2,
                si,
            )

        def gather(h):
            return pltpu.make_async_copy(
                sscores.at[idx1h[h]],
                vals[h].at[pl.ds(0, _LH[h] * _CCH)] if h == 1 else vals[h],
                sg[h],
            )

        def repack(h):
            dst = idx1h[h]

            def body(j, carry):
                for k in range(_CCH // 16):
                    dst[pl.ds(j * _CCH + k * 16, 16)] = idx2[
                        j, pl.ds(k * 16, 16)
                    ]
                return carry

            lax.fori_loop(0, _LH[h], body, 0, unroll=8)

        idx_dma(0, 0).start()

        def rd(kk, buf):
            off = sid * st_per_tile + st_offs[kk]
            return pltpu.make_async_copy(
                scores_hbm.at[pl.ds(off, st_sizes[kk])],
                vals[buf].at[pl.ds(0, st_sizes[kk])],
                sg[buf],
            )

        rd(0, 0).start()
        for kk in range(len(st_sizes)):
            bb = kk % 2
            if kk + 1 < len(st_sizes):
                rd(kk + 1, 1 - bb).start()
            rd(kk, bb).wait()
            off = sid * st_per_tile + st_offs[kk]
            pltpu.sync_copy(
                vals[bb].at[pl.ds(0, st_sizes[kk])],
                sscores.at[pl.ds(off, st_sizes[kk])],
            )

        @pl.when(sid == 0)
        def _stage_tail():
            toff = 16 * st_per_tile
            pltpu.sync_copy(
                scores_hbm.at[pl.ds(toff, st_tail)],
                val0.at[pl.ds(0, st_tail)],
            )
            pltpu.sync_copy(
                val0.at[pl.ds(0, st_tail)],
                sscores.at[pl.ds(toff, st_tail)],
            )

        plsc.subcore_barrier()

        idx_dma(0, 0).wait()
        repack(0)
        gather(0).start()
        idx_dma(0, 1).start()

        def reduce_half(h, accs):
            for k in range(_CCH // 16):

                def rbody(l, acc):
                    return acc + vals[h][pl.ds(l * _CCH + k * 16, 16)]

                accs[k] = lax.fori_loop(0, _LH[h], rbody, accs[k], unroll=8)

        for ch in range(_NCHUNK):
            accs = [jnp.zeros((16,), jnp.float32)] * (_CCH // 16)
            idx_dma(ch, 1).wait()
            repack(1)
            gather(1).start()
            if ch + 1 < _NCHUNK:
                idx_dma(ch + 1, 0).start()

            gather(0).wait()
            reduce_half(0, accs)
            if ch + 1 < _NCHUNK:
                idx_dma(ch + 1, 0).wait()
                repack(0)
                gather(0).start()
                idx_dma(ch + 1, 1).start()

            gather(1).wait()
            reduce_half(1, accs)

            for k in range(_CCH // 16):
                z = accs[k] * (1.0 / L)
                em = jnp.exp(-jnp.abs(z))
                p = jnp.where(z >= 0.0, 1.0, em) / (1.0 + em)
                v = p * 10000.0
                n = v.astype(jnp.int32)
                nf = n.astype(jnp.float32)
                f = v - nf
                up = (f > 0.5) | ((f == 0.5) & ((n & 1) == 1))
                r = (nf + jnp.where(up, 1.0, 0.0)) / 10000.0
                outv[pl.ds(ch * _CCH + k * 16, 16)] = r

        pltpu.sync_copy(outv, out_hbm.at[pl.ds(col0, _RPW)])

    return gather_pool


_gather_pool = _make_gather_pool()


def kernel(x, table, W, b):
    xT = x.astype(jnp.int32).T
    scores = _compute_scores(table.T, W, b)
    return _gather_pool(xT, scores).reshape(B, 1)
